# baseline (speedup 1.0000x reference)
import functools

import jax
import jax.numpy as jnp
from jax import lax
from jax.experimental import pallas as pl
from jax.experimental.pallas import tpu as pltpu
from jax.experimental.pallas import tpu_sc as plsc

N = 10000
E = 640000
FEAT = 128
HID = 128
OUTD = 8

NC = 2
NS = 16
NW = NC * NS
L = 16

NPAD = 10240
RPW = NPAD // NW
EC = E // NW
ECH = 80
NCHK = EC // ECH
G = ECH // L
EP = E + 8 * ECH
ACH = 128
AG = ACH // L
NEG = -1e30

_CP = pltpu.CompilerParams(needs_layout_passes=False)
_mesh = plsc.VectorSubcoreMesh(
    core_axis_name="c", subcore_axis_name="s", num_cores=NC, num_subcores=NS)


def _wid():
  return lax.axis_index("s") * NC + lax.axis_index("c")


def _iota():
  return lax.iota(jnp.int32, L)


def _gather16(v, idx):
  dn = lax.GatherDimensionNumbers(
      offset_dims=(), collapsed_slice_dims=(0,), start_index_map=(0,))
  return lax.gather(v, idx.reshape(L, 1), dn, (1,),
                    mode=lax.GatherScatterMode.PROMISE_IN_BOUNDS)


def _seg_scan(d16, v16, is_max):
  kk, vv = plsc.sort_key_val(d16, v16)
  it = _iota()
  for sh in (1, 2, 4, 8):
    idx = jnp.maximum(it - sh, 0)
    kq = _gather16(kk, idx)
    vq = _gather16(vv, idx)
    eq = (it >= sh) & (kq == kk)
    if is_max:
      vv = jnp.where(eq, jnp.maximum(vv, vq), vv)
    else:
      vv = jnp.where(eq, vv + vq, vv)
  nxt = _gather16(kk, jnp.minimum(it + 1, L - 1))
  last = (it == L - 1) | (nxt != kk)
  return kk, vv, last


@functools.partial(
    pl.kernel, mesh=_mesh, compiler_params=_CP,
    out_type=jax.ShapeDtypeStruct((NPAD, HID), jnp.float32),
    scratch_types=[
        pltpu.VMEM((4, 80), jnp.int32),
        pltpu.VMEM((RPW, HID), jnp.float32),
        pltpu.SemaphoreType.DMA,
    ])
def _emb_gather(emb_hbm, ids_hbm, out_hbm, idx_v, rows_v, sem):
  w = _wid()
  for j in range(4):
    pltpu.sync_copy(ids_hbm.at[pl.ds(w * RPW + j * 80, 80)], idx_v.at[j])
  for j in range(4):
    pltpu.async_copy(emb_hbm.at[idx_v.at[j]],
                     rows_v.at[pl.ds(j * 80, 80)], sem).wait()
  pltpu.sync_copy(rows_v, out_hbm.at[pl.ds(w * RPW, RPW)])


@functools.partial(
    pl.kernel, mesh=_mesh, compiler_params=_CP,
    out_type=jax.ShapeDtypeStruct((E,), jnp.float32),
    scratch_types=[
        pltpu.VMEM((EC,), jnp.int32),
        pltpu.VMEM((EC,), jnp.int32),
        pltpu.VMEM((2, ECH, HID), jnp.float32),
        pltpu.VMEM((2, ECH, HID), jnp.float32),
        pltpu.VMEM((HID,), jnp.float32),
        pltpu.VMEM((2, ECH), jnp.float32),
        pltpu.SemaphoreType.DMA,
        pltpu.SemaphoreType.DMA,
    ])
def _score_k(xl, xr, att, srcr, dstr, score, srca, dsta, xlr, xrr, attb,
             scob, srow, sout):
  w = _wid()
  ebase = w * EC
  pltpu.sync_copy(att, attb)
  pltpu.sync_copy(srcr.at[pl.ds(ebase, EC)], srca)
  pltpu.sync_copy(dstr.at[pl.ds(ebase, EC)], dsta)
  it = _iota()

  def row_copies(j, p):
    c1 = pltpu.make_async_copy(
        xl.at[srca.at[pl.ds(j * ECH, ECH)]], xlr.at[p], srow)
    c2 = pltpu.make_async_copy(
        xr.at[dsta.at[pl.ds(j * ECH, ECH)]], xrr.at[p], srow)
    return c1, c2

  for c in row_copies(0, 0):
    c.start()

  def chunk(j, _):
    p = lax.rem(j, 2)

    @pl.when(j < NCHK - 1)
    def _():
      for c in row_copies(j + 1, 1 - p):
        c.start()

    for c in row_copies(j, p):
      c.wait()

    @pl.when(j >= 2)
    def _():
      pltpu.make_async_copy(scob.at[p], score.at[pl.ds(ebase, ECH)],
                            sout).wait()

    pv = jnp.full((L,), p, jnp.int32)
    attv = [attb[pl.ds(cv * L, L)] for cv in range(HID // L)]

    def grp(g, _):
      score_vec = jnp.zeros((L,), jnp.float32)
      for l in range(L):
        rv = g * L + jnp.full((L,), l, jnp.int32)
        acc = jnp.zeros((L,), jnp.float32)
        for cv in range(HID // L):
          cc = cv * L + it
          a = plsc.load_gather(xlr, [pv, rv, cc])
          b = plsc.load_gather(xrr, [pv, rv, cc])
          z = a + b
          zl = jnp.maximum(z, 0.2 * z)
          acc = acc + attv[cv] * zl
        red = jnp.sum(acc)
        score_vec = jnp.where(it == l, red, score_vec)
      plsc.store_scatter(scob, [pv, g * L + it], score_vec)
      return 0

    lax.fori_loop(0, G, grp, 0)
    pltpu.async_copy(scob.at[p], score.at[pl.ds(ebase + j * ECH, ECH)], sout)
    return 0

  lax.fori_loop(0, NCHK, chunk, 0)
  for _ in range(2):
    pltpu.make_async_copy(scob.at[0], score.at[pl.ds(ebase, ECH)],
                          sout).wait()


@functools.partial(
    pl.kernel, mesh=_mesh, compiler_params=_CP,
    out_type=jax.ShapeDtypeStruct((NW * NPAD,), jnp.float32),
    scratch_types=[
        pltpu.VMEM((NPAD,), jnp.float32),
        pltpu.VMEM((EC,), jnp.float32),
        pltpu.VMEM((EC,), jnp.int32),
    ])
def _segmax_k(score, dstr, m_priv, m_v, scoa, dsta):
  w = _wid()
  ebase = w * EC
  neg = jnp.full((L,), NEG, jnp.float32)
  pltpu.sync_copy(score.at[pl.ds(ebase, EC)], scoa)
  pltpu.sync_copy(dstr.at[pl.ds(ebase, EC)], dsta)

  def init(i, _):
    m_v[pl.ds(i * L, L)] = neg
    return 0

  lax.fori_loop(0, NPAD // L, init, 0)

  def grp(g, _):
    s16 = scoa[pl.ds(g * L, L)]
    d16 = dsta[pl.ds(g * L, L)]
    kk, vv, last = _seg_scan(d16, s16, is_max=True)
    cur = plsc.load_gather(m_v, [kk])
    plsc.store_scatter(m_v, [kk], jnp.maximum(cur, vv), mask=last)
    return 0

  lax.fori_loop(0, EC // L, grp, 0)
  pltpu.sync_copy(m_v, m_priv.at[pl.ds(w * NPAD, NPAD)])


def _make_combine(is_max):
  @functools.partial(
      pl.kernel, mesh=_mesh, compiler_params=_CP,
      out_type=jax.ShapeDtypeStruct((NPAD,), jnp.float32),
      scratch_types=[
          pltpu.VMEM((RPW,), jnp.float32),
          pltpu.VMEM((NW * RPW,), jnp.float32),
          pltpu.SemaphoreType.DMA,
      ])
  def _combine(priv, glob, acc, buf, sem):
    w = _wid()
    c0 = w * RPW
    for j in range(NW):
      pltpu.async_copy(priv.at[pl.ds(j * NPAD + c0, RPW)],
                       buf.at[pl.ds(j * RPW, RPW)], sem)
    for j in range(NW):
      pltpu.make_async_copy(priv.at[pl.ds(c0, RPW)],
                            buf.at[pl.ds(j * RPW, RPW)], sem).wait()

    def body(j, _):
      for v in range(RPW // L):
        a = acc[pl.ds(v * L, L)]
        b = buf[pl.ds(j * RPW + v * L, L)]
        acc[pl.ds(v * L, L)] = jnp.maximum(a, b) if is_max else a + b
      return 0

    for v in range(RPW // L):
      acc[pl.ds(v * L, L)] = buf[pl.ds(v * L, L)]
    lax.fori_loop(1, NW, body, 0)
    pltpu.sync_copy(acc, glob.at[pl.ds(c0, RPW)])

  return _combine


_combine_max = _make_combine(True)
_combine_add = _make_combine(False)


@functools.partial(
    pl.kernel, mesh=_mesh, compiler_params=_CP,
    out_type=[
        jax.ShapeDtypeStruct((E,), jnp.float32),
        jax.ShapeDtypeStruct((NW * NPAD,), jnp.float32),
    ],
    scratch_types=[
        pltpu.VMEM((NPAD,), jnp.float32),
        pltpu.VMEM((NPAD,), jnp.float32),
        pltpu.VMEM((EC,), jnp.float32),
        pltpu.VMEM((EC,), jnp.int32),
        pltpu.VMEM((EC,), jnp.float32),
    ])
def _expsum_k(score, dstr, m_glob, ex, s_priv, m_v, s_v, scoa, dsta, exa):
  w = _wid()
  ebase = w * EC
  pltpu.sync_copy(m_glob, m_v)
  pltpu.sync_copy(score.at[pl.ds(ebase, EC)], scoa)
  pltpu.sync_copy(dstr.at[pl.ds(ebase, EC)], dsta)
  zero = jnp.zeros((L,), jnp.float32)

  def init(i, _):
    s_v[pl.ds(i * L, L)] = zero
    return 0

  lax.fori_loop(0, NPAD // L, init, 0)

  def grp(g, _):
    s16 = scoa[pl.ds(g * L, L)]
    d16 = dsta[pl.ds(g * L, L)]
    mv = plsc.load_gather(m_v, [d16])
    e16 = jnp.exp(s16 - mv)
    exa[pl.ds(g * L, L)] = e16
    kk, vv, last = _seg_scan(d16, e16, is_max=False)
    cur = plsc.load_gather(s_v, [kk])
    plsc.store_scatter(s_v, [kk], cur + vv, mask=last)
    return 0

  lax.fori_loop(0, EC // L, grp, 0)
  pltpu.sync_copy(exa, ex.at[pl.ds(ebase, EC)])
  pltpu.sync_copy(s_v, s_priv.at[pl.ds(w * NPAD, NPAD)])


@functools.partial(
    pl.kernel, mesh=_mesh, compiler_params=_CP,
    out_type=jax.ShapeDtypeStruct((E,), jnp.float32),
    scratch_types=[
        pltpu.VMEM((NPAD,), jnp.float32),
        pltpu.VMEM((EC,), jnp.float32),
        pltpu.VMEM((EC,), jnp.int32),
    ])
def _alpha_k(ex, dstr, s_glob, alpha, s_v, exa, dsta):
  w = _wid()
  ebase = w * EC
  pltpu.sync_copy(s_glob, s_v)
  pltpu.sync_copy(ex.at[pl.ds(ebase, EC)], exa)
  pltpu.sync_copy(dstr.at[pl.ds(ebase, EC)], dsta)

  def grp(g, _):
    e16 = exa[pl.ds(g * L, L)]
    d16 = dsta[pl.ds(g * L, L)]
    sv = plsc.load_gather(s_v, [d16])
    exa[pl.ds(g * L, L)] = e16 / (sv + 1e-16)
    return 0

  lax.fori_loop(0, EC // L, grp, 0)
  pltpu.sync_copy(exa, alpha.at[pl.ds(ebase, EC)])


@functools.partial(
    pl.kernel, mesh=_mesh, compiler_params=_CP,
    out_type=jax.ShapeDtypeStruct((NW * NPAD,), jnp.float32),
    scratch_types=[
        pltpu.VMEM((NPAD,), jnp.float32),
        pltpu.VMEM((EC,), jnp.int32),
    ])
def _hist_k(dstr, hist_priv, h_v, dsta):
  w = _wid()
  ebase = w * EC
  pltpu.sync_copy(dstr.at[pl.ds(ebase, EC)], dsta)
  zero = jnp.zeros((L,), jnp.float32)

  def init(i, _):
    h_v[pl.ds(i * L, L)] = zero
    return 0

  lax.fori_loop(0, NPAD // L, init, 0)
  ones = jnp.ones((L,), jnp.float32)

  def grp(g, _):
    d16 = dsta[pl.ds(g * L, L)]
    kk, vv, last = _seg_scan(d16, ones, is_max=False)
    cur = plsc.load_gather(h_v, [kk])
    plsc.store_scatter(h_v, [kk], cur + vv, mask=last)
    return 0

  lax.fori_loop(0, EC // L, grp, 0)
  pltpu.sync_copy(h_v, hist_priv.at[pl.ds(w * NPAD, NPAD)])


@functools.partial(
    pl.kernel, mesh=_mesh, compiler_params=_CP,
    out_type=jax.ShapeDtypeStruct((NW * 8,), jnp.float32),
    scratch_types=[
        pltpu.VMEM((NW * RPW,), jnp.float32),
        pltpu.VMEM((L,), jnp.float32),
        pltpu.SemaphoreType.DMA,
    ])
def _slicesum_k(hist_priv, ssum, buf, sb, sem):
  w = _wid()
  c0 = w * RPW
  for j in range(NW):
    pltpu.async_copy(hist_priv.at[pl.ds(j * NPAD + c0, RPW)],
                     buf.at[pl.ds(j * RPW, RPW)], sem)
  for j in range(NW):
    pltpu.make_async_copy(hist_priv.at[pl.ds(c0, RPW)],
                          buf.at[pl.ds(j * RPW, RPW)], sem).wait()
  acc = jnp.zeros((L,), jnp.float32)

  def body(i, a):
    return a + buf[pl.ds(i * L, L)]

  acc = lax.fori_loop(0, (NW * RPW) // L, body, acc)
  tot = jnp.sum(acc)
  it = _iota()
  sb[pl.ds(0, L)] = jnp.where(it == 0, tot, 0.0)
  pltpu.sync_copy(sb.at[pl.ds(0, 8)], ssum.at[pl.ds(w * 8, 8)])


@functools.partial(
    pl.kernel, mesh=_mesh, compiler_params=_CP,
    out_type=[
        jax.ShapeDtypeStruct((NW * NPAD,), jnp.float32),
        jax.ShapeDtypeStruct((NPAD + 8,), jnp.float32),
    ],
    scratch_types=[
        pltpu.VMEM((NW * RPW,), jnp.float32),
        pltpu.VMEM((NW * 8,), jnp.float32),
        pltpu.VMEM((NW,), jnp.float32),
        pltpu.VMEM((RPW,), jnp.float32),
        pltpu.VMEM((RPW,), jnp.float32),
        pltpu.VMEM((L,), jnp.float32),
        pltpu.SemaphoreType.DMA,
    ])
def _base_k(hist_priv, ssum, base, start, buf, ssv, pv_, startv, bb, eb, sem):
  w = _wid()
  c0 = w * RPW
  for j in range(NW):
    pltpu.async_copy(hist_priv.at[pl.ds(j * NPAD + c0, RPW)],
                     buf.at[pl.ds(j * RPW, RPW)], sem)
  pltpu.sync_copy(ssum, ssv)
  it = _iota()
  idx8 = it * 8
  sv0 = plsc.load_gather(ssv, [idx8])
  cs0 = plsc.cumsum(sv0)
  pv_[pl.ds(0, L)] = cs0 - sv0
  sv1 = plsc.load_gather(ssv, [idx8 + L * 8])
  cs1 = plsc.cumsum(sv1)
  pv_[pl.ds(L, L)] = cs1 - sv1 + cs0[L - 1]
  my_start = plsc.load_gather(pv_, [jnp.full((L,), 1, jnp.int32) * w])[0]

  for j in range(NW):
    pltpu.make_async_copy(hist_priv.at[pl.ds(c0, RPW)],
                          buf.at[pl.ds(j * RPW, RPW)], sem).wait()

  def totb(i, _):
    a = jnp.zeros((L,), jnp.float32)
    for j in range(NW):
      a = a + buf[pl.ds(j * RPW + i * L, L)]
    startv[pl.ds(i * L, L)] = a
    return 0

  lax.fori_loop(0, RPW // L, totb, 0)
  carry2 = my_start
  for v in range(RPW // L):
    tv = startv[pl.ds(v * L, L)]
    cs = plsc.cumsum(tv)
    startv[pl.ds(v * L, L)] = cs - tv + carry2
    carry2 = carry2 + cs[L - 1]
  pltpu.sync_copy(startv, start.at[pl.ds(c0, RPW)])

  @pl.when(w == NW - 1)
  def _():
    eb[pl.ds(0, L)] = jnp.full((L,), float(E), jnp.float32)
    pltpu.sync_copy(eb.at[pl.ds(0, 8)], start.at[pl.ds(NPAD, 8)])

  for v in range(RPW // L):
    bb[pl.ds(v * L, L)] = startv[pl.ds(v * L, L)]

  def tbody(t, _):
    pltpu.sync_copy(bb, base.at[pl.ds(t * NPAD + c0, RPW)])
    for v in range(RPW // L):
      bb[pl.ds(v * L, L)] = (bb[pl.ds(v * L, L)] +
                             buf[pl.ds(t * RPW + v * L, L)])
    return 0

  lax.fori_loop(0, NW, tbody, 0)


@functools.partial(
    pl.kernel, mesh=_mesh, compiler_params=_CP,
    out_type=[
        jax.ShapeDtypeStruct((EP,), jnp.int32),
        jax.ShapeDtypeStruct((EP,), jnp.int32),
        jax.ShapeDtypeStruct((EP,), jnp.int32),
    ],
    scratch_types=[
        pltpu.VMEM((NPAD,), jnp.float32),
        pltpu.VMEM((EC,), jnp.int32),
        pltpu.VMEM((EC,), jnp.int32),
        pltpu.VMEM((2, ECH), jnp.int32),
        pltpu.VMEM((2, ECH), jnp.int32),
        pltpu.VMEM((2, ECH), jnp.int32),
        pltpu.VMEM((2, ECH), jnp.int32),
        pltpu.VMEM((L,), jnp.int32),
        pltpu.SemaphoreType.DMA,
    ])
def _place_k(srcr, dstr, base, s_src, s_dst, perm, bw, srca, dsta, posb,
             srb, drb, eib, zb, sem):
  w = _wid()
  ebase = w * EC
  pltpu.sync_copy(base.at[pl.ds(w * NPAD, NPAD)], bw)
  pltpu.sync_copy(srcr.at[pl.ds(ebase, EC)], srca)
  pltpu.sync_copy(dstr.at[pl.ds(ebase, EC)], dsta)
  it = _iota()
  ones = jnp.ones((L,), jnp.float32)

  def sc_copies(p):
    return (
        pltpu.make_async_copy(srb.at[p], s_src.at[posb.at[p]], sem),
        pltpu.make_async_copy(drb.at[p], s_dst.at[posb.at[p]], sem),
        pltpu.make_async_copy(eib.at[p], perm.at[posb.at[p]], sem),
    )

  def chunk(j, _):
    p = lax.rem(j, 2)

    @pl.when(j >= 2)
    def _():
      for c in sc_copies(p):
        c.wait()

    pv = jnp.full((L,), p, jnp.int32)
    for g in range(G):
      d16 = plsc.load_gather(dsta, [j * ECH + g * L + it])
      s16 = plsc.load_gather(srca, [j * ECH + g * L + it])
      kk, lane = plsc.sort_key_val(d16, it)
      vv = ones
      for sh in (1, 2, 4, 8):
        idx = jnp.maximum(it - sh, 0)
        kq = _gather16(kk, idx)
        vq = _gather16(vv, idx)
        eq = (it >= sh) & (kq == kk)
        vv = jnp.where(eq, vv + vq, vv)
      nxt = _gather16(kk, jnp.minimum(it + 1, L - 1))
      last = (it == L - 1) | (nxt != kk)
      cur = plsc.load_gather(bw, [kk])
      pos16 = (cur + vv - 1.0).astype(jnp.int32)
      plsc.store_scatter(bw, [kk], cur + vv, mask=last)
      src_s = _gather16(s16, lane)
      eid = ebase + j * ECH + g * L + lane
      cvec = g * L + it
      plsc.store_scatter(posb, [pv, cvec], pos16)
      plsc.store_scatter(srb, [pv, cvec], src_s)
      plsc.store_scatter(drb, [pv, cvec], kk)
      plsc.store_scatter(eib, [pv, cvec], eid)
    for c in sc_copies(p):
      c.start()
    return 0

  lax.fori_loop(0, NCHK, chunk, 0)
  for p in range(2):
    for c in sc_copies(p):
      c.wait()

  @pl.when(w == NW - 1)
  def _():
    zb[pl.ds(0, L)] = jnp.zeros((L,), jnp.int32)

    def padb(i, _):
      pltpu.sync_copy(zb, s_src.at[pl.ds(E + i * L, L)])
      pltpu.sync_copy(zb, s_dst.at[pl.ds(E + i * L, L)])
      pltpu.sync_copy(zb, perm.at[pl.ds(E + i * L, L)])
      return 0

    lax.fori_loop(0, (EP - E) // L, padb, 0)


@functools.partial(
    pl.kernel, mesh=_mesh, compiler_params=_CP,
    out_type=jax.ShapeDtypeStruct((NPAD, HID), jnp.float32),
    scratch_types=[
        pltpu.VMEM((RPW, HID), jnp.float32),
        pltpu.VMEM((2, ACH, HID), jnp.float32),
        pltpu.VMEM((2, ACH), jnp.int32),
        pltpu.VMEM((2, ACH), jnp.int32),
        pltpu.VMEM((2, ACH), jnp.int32),
        pltpu.VMEM((2, ACH), jnp.float32),
        pltpu.VMEM((RPW + 8,), jnp.float32),
        pltpu.SemaphoreType.DMA,
        pltpu.SemaphoreType.DMA,
    ])
def _aggcsr_k(alpha, perm, s_src, s_dst, start, xl, zeros, out, out_buf,
              rows, srcb, dstb, permb, alb, startv, srow, sidx):
  w = _wid()
  c0 = w * RPW
  pltpu.sync_copy(zeros.at[pl.ds(0, RPW)], out_buf)
  pltpu.sync_copy(start.at[pl.ds(c0, RPW + 8)], startv)
  it = _iota()
  lo = startv[pl.ds(0, L)][0].astype(jnp.int32)
  hi = startv[pl.ds(RPW - 8, L)][8].astype(jnp.int32)
  lo8 = pl.multiple_of(lo - lax.rem(lo, 8), 8)
  nch = (hi - lo8 + (ACH - 1)) // ACH

  def idx_copies(j, p):
    e0 = pl.multiple_of(lo8 + j * ACH, 8)
    return (
        pltpu.make_async_copy(s_src.at[pl.ds(e0, ACH)], srcb.at[p], sidx),
        pltpu.make_async_copy(s_dst.at[pl.ds(e0, ACH)], dstb.at[p], sidx),
        pltpu.make_async_copy(perm.at[pl.ds(e0, ACH)], permb.at[p], sidx),
    )

  def row_copies(j, p):
    return (
        pltpu.make_async_copy(xl.at[srcb.at[p]], rows.at[p], srow),
        pltpu.make_async_copy(alpha.at[permb.at[p]], alb.at[p], srow),
    )

  @pl.when(nch > 0)
  def _():
    for c in idx_copies(0, 0):
      c.start()
      c.wait()
    for c in row_copies(0, 0):
      c.start()

  def chunk(j, _):
    p = lax.rem(j, 2)

    @pl.when(j < nch - 1)
    def _():
      for c in idx_copies(j + 1, 1 - p):
        c.start()

    for c in row_copies(j, p):
      c.wait()

    pv = jnp.full((L,), p, jnp.int32)
    base_pos = lo8 + j * ACH
    for g in range(AG):
      cvec = g * L + it
      p16 = base_pos + g * L + it
      valid = (p16 >= lo) & (p16 < hi)
      a16 = plsc.load_gather(alb, [pv, cvec])
      d16 = plsc.load_gather(dstb, [pv, cvec])
      a_eff = jnp.where(valid, a16, 0.0)
      dl_eff = jnp.where(valid, d16 - c0, 0)
      for lb in range(0, L, 4):
        xs = []
        for l in range(lb, lb + 4):
          rv = g * L + jnp.full((L,), l, jnp.int32)
          a_sc = a_eff[l]
          for cv in range(HID // L):
            cc = cv * L + it
            xs.append(plsc.load_gather(rows, [pv, rv, cc]) * a_sc)
        k = 0
        for l in range(lb, lb + 4):
          dlv = jnp.full((L,), 1, jnp.int32) * dl_eff[l]
          for cv in range(HID // L):
            cc = cv * L + it
            plsc.addupdate_scatter(out_buf, [dlv, cc], xs[k])
            k += 1

    @pl.when(j < nch - 1)
    def _():
      for c in idx_copies(j + 1, 1 - p):
        c.wait()
      for c in row_copies(j + 1, 1 - p):
        c.start()

    return 0

  lax.fori_loop(0, nch, chunk, 0)
  pltpu.sync_copy(out_buf, out.at[pl.ds(c0, RPW)])


_BR = 512


def _combine_body(g_ref, xp_ref, wa_ref, wb_ref, bp_ref, o_ref):
  acc = jnp.dot(g_ref[...], wa_ref[...], preferred_element_type=jnp.float32)
  acc += jnp.dot(xp_ref[...], wb_ref[...], preferred_element_type=jnp.float32)
  o_ref[...] = jnp.maximum(acc + bp_ref[...], 0.0)


def _tc_combine(g, xp, wa, wb, bp):
  return pl.pallas_call(
      _combine_body,
      grid=(NPAD // _BR,),
      in_specs=[
          pl.BlockSpec((_BR, HID), lambda i: (i, 0)),
          pl.BlockSpec((_BR, FEAT), lambda i: (i, 0)),
          pl.BlockSpec((HID, HID), lambda i: (0, 0)),
          pl.BlockSpec((FEAT, HID), lambda i: (0, 0)),
          pl.BlockSpec((1, HID), lambda i: (0, 0)),
      ],
      out_specs=pl.BlockSpec((_BR, HID), lambda i: (i, 0)),
      out_shape=jax.ShapeDtypeStruct((NPAD, HID), jnp.float32),
  )(g, xp, wa, wb, bp)


def _proj_body(x_ref, w_ref, o_ref):
  o_ref[0] = jnp.dot(x_ref[...], w_ref[...],
                     preferred_element_type=jnp.float32)


def _tc_proj_heads(x, w, heads):
  return pl.pallas_call(
      _proj_body,
      grid=(heads, NPAD // _BR),
      in_specs=[
          pl.BlockSpec((_BR, HID), lambda h, i: (i, 0)),
          pl.BlockSpec((HID, HID), lambda h, i: (0, h)),
      ],
      out_specs=pl.BlockSpec((1, _BR, HID), lambda h, i: (h, i, 0)),
      out_shape=jax.ShapeDtypeStruct((heads, NPAD, HID), jnp.float32),
  )(x, w)


def _layer2in_body(p_ref, b1_ref, wl_ref, wr_ref, ol_ref, or_ref):
  accl = jnp.zeros((_BR, HID), jnp.float32)
  accr = jnp.zeros((_BR, HID), jnp.float32)
  for h in range(4):
    xh = jnp.maximum(p_ref[h] + b1_ref[0, pl.ds(h * HID, HID)], 0.0)
    accl += jnp.dot(xh, wl_ref[pl.ds(h * HID, HID), :],
                    preferred_element_type=jnp.float32)
    accr += jnp.dot(xh, wr_ref[pl.ds(h * HID, HID), :],
                    preferred_element_type=jnp.float32)
  ol_ref[...] = accl
  or_ref[...] = accr


def _tc_layer2in(p, b1, wl, wr):
  return pl.pallas_call(
      _layer2in_body,
      grid=(NPAD // _BR,),
      in_specs=[
          pl.BlockSpec((4, _BR, HID), lambda i: (0, i, 0)),
          pl.BlockSpec((1, 4 * HID), lambda i: (0, 0)),
          pl.BlockSpec((4 * HID, HID), lambda i: (0, 0)),
          pl.BlockSpec((4 * HID, HID), lambda i: (0, 0)),
      ],
      out_specs=[
          pl.BlockSpec((_BR, HID), lambda i: (i, 0)),
          pl.BlockSpec((_BR, HID), lambda i: (i, 0)),
      ],
      out_shape=[
          jax.ShapeDtypeStruct((NPAD, HID), jnp.float32),
          jax.ShapeDtypeStruct((NPAD, HID), jnp.float32),
      ],
  )(p, b1, wl, wr)


def _final_body(p_ref, b2_ref, wo_ref, bo_ref, o_ref):
  xo = p_ref[...] + b2_ref[...]
  o_ref[...] = jnp.dot(xo, wo_ref[...],
                       preferred_element_type=jnp.float32) + bo_ref[...]


def _tc_final(p, b2, wo, bo):
  return pl.pallas_call(
      _final_body,
      grid=(NPAD // _BR,),
      in_specs=[
          pl.BlockSpec((_BR, HID), lambda i: (i, 0)),
          pl.BlockSpec((1, HID), lambda i: (0, 0)),
          pl.BlockSpec((HID, HID), lambda i: (0, 0)),
          pl.BlockSpec((1, HID), lambda i: (0, 0)),
      ],
      out_specs=pl.BlockSpec((_BR, HID), lambda i: (i, 0)),
      out_shape=jax.ShapeDtypeStruct((NPAD, HID), jnp.float32),
  )(p, b2, wo, bo)


def _gat_head(xl_h, xr_h, att_h, src, dst, zeros, csr):
  s_src, s_dst, perm, start = csr
  score = _score_k(xl_h, xr_h, att_h, src, dst)
  m_priv = _segmax_k(score, dst)
  m_glob = _combine_max(m_priv)
  ex, s_priv = _expsum_k(score, dst, m_glob)
  s_glob = _combine_add(s_priv)
  alpha = _alpha_k(ex, dst, s_glob)
  return _aggcsr_k(alpha, perm, s_src, s_dst, start, xl_h, zeros)


def kernel(x_player, edge_index, emb, Wp, bp, Wl1, Wr1, att1, b1, Wl2, Wr2,
           att2, b2, Wo, bo):
  src = edge_index[0].astype(jnp.int32)
  dst = edge_index[1].astype(jnp.int32)

  ids = x_player[:, 0].astype(jnp.int32)
  ids_pad = jnp.pad(ids, (0, NPAD - N))
  xp = jnp.pad(x_player[:, 1:], ((0, NPAD - N), (0, 0)))

  g = _emb_gather(emb, ids_pad)
  x0 = _tc_combine(g, xp, Wp[:HID], Wp[HID:], bp.reshape(1, HID))

  xl1 = _tc_proj_heads(x0, Wl1, 4)
  xr1 = _tc_proj_heads(x0, Wr1, 4)
  zeros = jnp.zeros((NPAD, HID), jnp.float32)
  hist = _hist_k(dst)
  ssum = _slicesum_k(hist)
  base, start = _base_k(hist, ssum)
  s_src, s_dst, perm = _place_k(src, dst, base)
  csr = (s_src, s_dst, perm, start)
  parts1 = [_gat_head(xl1[h], xr1[h], att1[h], src, dst, zeros, csr)
            for h in range(4)]
  p1 = jnp.stack(parts1, axis=0)

  xl2, xr2 = _tc_layer2in(p1, b1.reshape(1, 4 * HID), Wl2, Wr2)
  p2 = _gat_head(xl2, xr2, att2[0], src, dst, zeros, csr)

  wo_pad = jnp.pad(Wo, ((0, 0), (0, HID - OUTD)))
  bo_pad = jnp.pad(bo, (0, HID - OUTD)).reshape(1, HID)
  y = _tc_final(p2, b2.reshape(1, HID), wo_pad, bo_pad)
  return y[:N, :OUTD]


# speedup vs baseline: 1.1884x; 1.1884x over previous
import functools

import jax
import jax.numpy as jnp
from jax import lax
from jax.experimental import pallas as pl
from jax.experimental.pallas import tpu as pltpu
from jax.experimental.pallas import tpu_sc as plsc

N = 10000
E = 640000
FEAT = 128
HID = 128
OUTD = 8

NC = 2
NS = 16
NW = NC * NS
L = 16

NPAD = 10240
RPW = NPAD // NW
EC = E // NW
ECH = 80
NCHK = EC // ECH
G = ECH // L
EP = E + 8 * ECH
NEG = -1e30

_CP = pltpu.CompilerParams(needs_layout_passes=False)
_mesh = plsc.VectorSubcoreMesh(
    core_axis_name="c", subcore_axis_name="s", num_cores=NC, num_subcores=NS)


def _wid():
  return lax.axis_index("s") * NC + lax.axis_index("c")


def _iota():
  return lax.iota(jnp.int32, L)


def _gather16(v, idx):
  dn = lax.GatherDimensionNumbers(
      offset_dims=(), collapsed_slice_dims=(0,), start_index_map=(0,))
  return lax.gather(v, idx.reshape(L, 1), dn, (1,),
                    mode=lax.GatherScatterMode.PROMISE_IN_BOUNDS)


def _seg_scan(d16, v16, is_max):
  kk, vv = plsc.sort_key_val(d16, v16)
  it = _iota()
  for sh in (1, 2, 4, 8):
    idx = jnp.maximum(it - sh, 0)
    kq = _gather16(kk, idx)
    vq = _gather16(vv, idx)
    eq = (it >= sh) & (kq == kk)
    if is_max:
      vv = jnp.where(eq, jnp.maximum(vv, vq), vv)
    else:
      vv = jnp.where(eq, vv + vq, vv)
  nxt = _gather16(kk, jnp.minimum(it + 1, L - 1))
  last = (it == L - 1) | (nxt != kk)
  return kk, vv, last


@functools.partial(
    pl.kernel, mesh=_mesh, compiler_params=_CP,
    out_type=jax.ShapeDtypeStruct((NPAD, HID), jnp.float32),
    scratch_types=[
        pltpu.VMEM((4, 80), jnp.int32),
        pltpu.VMEM((RPW, HID), jnp.float32),
        pltpu.SemaphoreType.DMA,
    ])
def _emb_gather(emb_hbm, ids_hbm, out_hbm, idx_v, rows_v, sem):
  w = _wid()
  for j in range(4):
    pltpu.sync_copy(ids_hbm.at[pl.ds(w * RPW + j * 80, 80)], idx_v.at[j])
  for j in range(4):
    pltpu.async_copy(emb_hbm.at[idx_v.at[j]],
                     rows_v.at[pl.ds(j * 80, 80)], sem).wait()
  pltpu.sync_copy(rows_v, out_hbm.at[pl.ds(w * RPW, RPW)])


@functools.partial(
    pl.kernel, mesh=_mesh, compiler_params=_CP,
    out_type=jax.ShapeDtypeStruct((E,), jnp.float32),
    scratch_types=[
        pltpu.VMEM((EC,), jnp.int32),
        pltpu.VMEM((EC,), jnp.int32),
        pltpu.VMEM((2, ECH, HID), jnp.float32),
        pltpu.VMEM((2, ECH, HID), jnp.float32),
        pltpu.VMEM((HID,), jnp.float32),
        pltpu.VMEM((2, ECH), jnp.float32),
        pltpu.SemaphoreType.DMA,
        pltpu.SemaphoreType.DMA,
    ])
def _score_k(xl, xr, att, srcr, dstr, score, srca, dsta, xlr, xrr, attb,
             scob, srow, sout):
  w = _wid()
  ebase = w * EC
  pltpu.sync_copy(att, attb)
  pltpu.sync_copy(srcr.at[pl.ds(ebase, EC)], srca)
  pltpu.sync_copy(dstr.at[pl.ds(ebase, EC)], dsta)
  it = _iota()

  def row_copies(j, p):
    c1 = pltpu.make_async_copy(
        xl.at[srca.at[pl.ds(j * ECH, ECH)]], xlr.at[p], srow)
    c2 = pltpu.make_async_copy(
        xr.at[dsta.at[pl.ds(j * ECH, ECH)]], xrr.at[p], srow)
    return c1, c2

  for c in row_copies(0, 0):
    c.start()

  def chunk(j, _):
    p = lax.rem(j, 2)

    @pl.when(j < NCHK - 1)
    def _():
      for c in row_copies(j + 1, 1 - p):
        c.start()

    for c in row_copies(j, p):
      c.wait()

    @pl.when(j >= 2)
    def _():
      pltpu.make_async_copy(scob.at[p], score.at[pl.ds(ebase, ECH)],
                            sout).wait()

    pv = jnp.full((L,), p, jnp.int32)
    attv = [attb[pl.ds(cv * L, L)] for cv in range(HID // L)]

    def grp(g, _):
      score_vec = jnp.zeros((L,), jnp.float32)
      for l in range(L):
        rv = g * L + jnp.full((L,), l, jnp.int32)
        acc = jnp.zeros((L,), jnp.float32)
        for cv in range(HID // L):
          cc = cv * L + it
          a = plsc.load_gather(xlr, [pv, rv, cc])
          b = plsc.load_gather(xrr, [pv, rv, cc])
          z = a + b
          zl = jnp.maximum(z, 0.2 * z)
          acc = acc + attv[cv] * zl
        red = jnp.sum(acc)
        score_vec = jnp.where(it == l, red, score_vec)
      plsc.store_scatter(scob, [pv, g * L + it], score_vec)
      return 0

    lax.fori_loop(0, G, grp, 0)
    pltpu.async_copy(scob.at[p], score.at[pl.ds(ebase + j * ECH, ECH)], sout)
    return 0

  lax.fori_loop(0, NCHK, chunk, 0)
  for _ in range(2):
    pltpu.make_async_copy(scob.at[0], score.at[pl.ds(ebase, ECH)],
                          sout).wait()


@functools.partial(
    pl.kernel, mesh=_mesh, compiler_params=_CP,
    out_type=jax.ShapeDtypeStruct((NW * NPAD,), jnp.float32),
    scratch_types=[
        pltpu.VMEM((NPAD,), jnp.float32),
        pltpu.VMEM((EC,), jnp.float32),
        pltpu.VMEM((EC,), jnp.int32),
    ])
def _segmax_k(score, dstr, m_priv, m_v, scoa, dsta):
  w = _wid()
  ebase = w * EC
  neg = jnp.full((L,), NEG, jnp.float32)
  pltpu.sync_copy(score.at[pl.ds(ebase, EC)], scoa)
  pltpu.sync_copy(dstr.at[pl.ds(ebase, EC)], dsta)

  def init(i, _):
    m_v[pl.ds(i * L, L)] = neg
    return 0

  lax.fori_loop(0, NPAD // L, init, 0)

  def grp(g, _):
    s16 = scoa[pl.ds(g * L, L)]
    d16 = dsta[pl.ds(g * L, L)]
    kk, vv, last = _seg_scan(d16, s16, is_max=True)
    cur = plsc.load_gather(m_v, [kk])
    plsc.store_scatter(m_v, [kk], jnp.maximum(cur, vv), mask=last)
    return 0

  lax.fori_loop(0, EC // L, grp, 0)
  pltpu.sync_copy(m_v, m_priv.at[pl.ds(w * NPAD, NPAD)])


def _make_combine(is_max):
  @functools.partial(
      pl.kernel, mesh=_mesh, compiler_params=_CP,
      out_type=jax.ShapeDtypeStruct((NPAD,), jnp.float32),
      scratch_types=[
          pltpu.VMEM((RPW,), jnp.float32),
          pltpu.VMEM((NW * RPW,), jnp.float32),
          pltpu.SemaphoreType.DMA,
      ])
  def _combine(priv, glob, acc, buf, sem):
    w = _wid()
    c0 = w * RPW
    for j in range(NW):
      pltpu.async_copy(priv.at[pl.ds(j * NPAD + c0, RPW)],
                       buf.at[pl.ds(j * RPW, RPW)], sem)
    for j in range(NW):
      pltpu.make_async_copy(priv.at[pl.ds(c0, RPW)],
                            buf.at[pl.ds(j * RPW, RPW)], sem).wait()

    def body(j, _):
      for v in range(RPW // L):
        a = acc[pl.ds(v * L, L)]
        b = buf[pl.ds(j * RPW + v * L, L)]
        acc[pl.ds(v * L, L)] = jnp.maximum(a, b) if is_max else a + b
      return 0

    for v in range(RPW // L):
      acc[pl.ds(v * L, L)] = buf[pl.ds(v * L, L)]
    lax.fori_loop(1, NW, body, 0)
    pltpu.sync_copy(acc, glob.at[pl.ds(c0, RPW)])

  return _combine


_combine_max = _make_combine(True)
_combine_add = _make_combine(False)


@functools.partial(
    pl.kernel, mesh=_mesh, compiler_params=_CP,
    out_type=[
        jax.ShapeDtypeStruct((E,), jnp.float32),
        jax.ShapeDtypeStruct((NW * NPAD,), jnp.float32),
    ],
    scratch_types=[
        pltpu.VMEM((NPAD,), jnp.float32),
        pltpu.VMEM((NPAD,), jnp.float32),
        pltpu.VMEM((EC,), jnp.float32),
        pltpu.VMEM((EC,), jnp.int32),
        pltpu.VMEM((EC,), jnp.float32),
    ])
def _expsum_k(score, dstr, m_glob, ex, s_priv, m_v, s_v, scoa, dsta, exa):
  w = _wid()
  ebase = w * EC
  pltpu.sync_copy(m_glob, m_v)
  pltpu.sync_copy(score.at[pl.ds(ebase, EC)], scoa)
  pltpu.sync_copy(dstr.at[pl.ds(ebase, EC)], dsta)
  zero = jnp.zeros((L,), jnp.float32)

  def init(i, _):
    s_v[pl.ds(i * L, L)] = zero
    return 0

  lax.fori_loop(0, NPAD // L, init, 0)

  def grp(g, _):
    s16 = scoa[pl.ds(g * L, L)]
    d16 = dsta[pl.ds(g * L, L)]
    mv = plsc.load_gather(m_v, [d16])
    e16 = jnp.exp(s16 - mv)
    exa[pl.ds(g * L, L)] = e16
    kk, vv, last = _seg_scan(d16, e16, is_max=False)
    cur = plsc.load_gather(s_v, [kk])
    plsc.store_scatter(s_v, [kk], cur + vv, mask=last)
    return 0

  lax.fori_loop(0, EC // L, grp, 0)
  pltpu.sync_copy(exa, ex.at[pl.ds(ebase, EC)])
  pltpu.sync_copy(s_v, s_priv.at[pl.ds(w * NPAD, NPAD)])


@functools.partial(
    pl.kernel, mesh=_mesh, compiler_params=_CP,
    out_type=jax.ShapeDtypeStruct((E,), jnp.float32),
    scratch_types=[
        pltpu.VMEM((NPAD,), jnp.float32),
        pltpu.VMEM((EC,), jnp.float32),
        pltpu.VMEM((EC,), jnp.int32),
    ])
def _alpha_k(ex, dstr, s_glob, alpha, s_v, exa, dsta):
  w = _wid()
  ebase = w * EC
  pltpu.sync_copy(s_glob, s_v)
  pltpu.sync_copy(ex.at[pl.ds(ebase, EC)], exa)
  pltpu.sync_copy(dstr.at[pl.ds(ebase, EC)], dsta)

  def grp(g, _):
    e16 = exa[pl.ds(g * L, L)]
    d16 = dsta[pl.ds(g * L, L)]
    sv = plsc.load_gather(s_v, [d16])
    exa[pl.ds(g * L, L)] = e16 / (sv + 1e-16)
    return 0

  lax.fori_loop(0, EC // L, grp, 0)
  pltpu.sync_copy(exa, alpha.at[pl.ds(ebase, EC)])


@functools.partial(
    pl.kernel, mesh=_mesh, compiler_params=_CP,
    out_type=jax.ShapeDtypeStruct((NW * NPAD,), jnp.float32),
    scratch_types=[
        pltpu.VMEM((NPAD,), jnp.float32),
        pltpu.VMEM((EC,), jnp.int32),
    ])
def _hist_k(dstr, hist_priv, h_v, dsta):
  w = _wid()
  ebase = w * EC
  pltpu.sync_copy(dstr.at[pl.ds(ebase, EC)], dsta)
  zero = jnp.zeros((L,), jnp.float32)

  def init(i, _):
    h_v[pl.ds(i * L, L)] = zero
    return 0

  lax.fori_loop(0, NPAD // L, init, 0)
  ones = jnp.ones((L,), jnp.float32)

  def grp(g, _):
    d16 = dsta[pl.ds(g * L, L)]
    kk, vv, last = _seg_scan(d16, ones, is_max=False)
    cur = plsc.load_gather(h_v, [kk])
    plsc.store_scatter(h_v, [kk], cur + vv, mask=last)
    return 0

  lax.fori_loop(0, EC // L, grp, 0)
  pltpu.sync_copy(h_v, hist_priv.at[pl.ds(w * NPAD, NPAD)])


@functools.partial(
    pl.kernel, mesh=_mesh, compiler_params=_CP,
    out_type=jax.ShapeDtypeStruct((NW * 8,), jnp.float32),
    scratch_types=[
        pltpu.VMEM((NW * RPW,), jnp.float32),
        pltpu.VMEM((L,), jnp.float32),
        pltpu.SemaphoreType.DMA,
    ])
def _slicesum_k(hist_priv, ssum, buf, sb, sem):
  w = _wid()
  c0 = w * RPW
  for j in range(NW):
    pltpu.async_copy(hist_priv.at[pl.ds(j * NPAD + c0, RPW)],
                     buf.at[pl.ds(j * RPW, RPW)], sem)
  for j in range(NW):
    pltpu.make_async_copy(hist_priv.at[pl.ds(c0, RPW)],
                          buf.at[pl.ds(j * RPW, RPW)], sem).wait()
  acc = jnp.zeros((L,), jnp.float32)

  def body(i, a):
    return a + buf[pl.ds(i * L, L)]

  acc = lax.fori_loop(0, (NW * RPW) // L, body, acc)
  tot = jnp.sum(acc)
  it = _iota()
  sb[pl.ds(0, L)] = jnp.where(it == 0, tot, 0.0)
  pltpu.sync_copy(sb.at[pl.ds(0, 8)], ssum.at[pl.ds(w * 8, 8)])


@functools.partial(
    pl.kernel, mesh=_mesh, compiler_params=_CP,
    out_type=[
        jax.ShapeDtypeStruct((NW * NPAD,), jnp.float32),
        jax.ShapeDtypeStruct((NPAD + 8,), jnp.float32),
    ],
    scratch_types=[
        pltpu.VMEM((NW * RPW,), jnp.float32),
        pltpu.VMEM((NW * 8,), jnp.float32),
        pltpu.VMEM((NW,), jnp.float32),
        pltpu.VMEM((RPW,), jnp.float32),
        pltpu.VMEM((RPW,), jnp.float32),
        pltpu.VMEM((L,), jnp.float32),
        pltpu.SemaphoreType.DMA,
    ])
def _base_k(hist_priv, ssum, base, start, buf, ssv, pv_, startv, bb, eb, sem):
  w = _wid()
  c0 = w * RPW
  for j in range(NW):
    pltpu.async_copy(hist_priv.at[pl.ds(j * NPAD + c0, RPW)],
                     buf.at[pl.ds(j * RPW, RPW)], sem)
  pltpu.sync_copy(ssum, ssv)
  it = _iota()
  idx8 = it * 8
  sv0 = plsc.load_gather(ssv, [idx8])
  cs0 = plsc.cumsum(sv0)
  pv_[pl.ds(0, L)] = cs0 - sv0
  sv1 = plsc.load_gather(ssv, [idx8 + L * 8])
  cs1 = plsc.cumsum(sv1)
  pv_[pl.ds(L, L)] = cs1 - sv1 + cs0[L - 1]
  my_start = plsc.load_gather(pv_, [jnp.full((L,), 1, jnp.int32) * w])[0]

  for j in range(NW):
    pltpu.make_async_copy(hist_priv.at[pl.ds(c0, RPW)],
                          buf.at[pl.ds(j * RPW, RPW)], sem).wait()

  def totb(i, _):
    a = jnp.zeros((L,), jnp.float32)
    for j in range(NW):
      a = a + buf[pl.ds(j * RPW + i * L, L)]
    startv[pl.ds(i * L, L)] = a
    return 0

  lax.fori_loop(0, RPW // L, totb, 0)
  carry2 = my_start
  for v in range(RPW // L):
    tv = startv[pl.ds(v * L, L)]
    cs = plsc.cumsum(tv)
    startv[pl.ds(v * L, L)] = cs - tv + carry2
    carry2 = carry2 + cs[L - 1]
  pltpu.sync_copy(startv, start.at[pl.ds(c0, RPW)])

  @pl.when(w == NW - 1)
  def _():
    eb[pl.ds(0, L)] = jnp.full((L,), float(E), jnp.float32)
    pltpu.sync_copy(eb.at[pl.ds(0, 8)], start.at[pl.ds(NPAD, 8)])

  for v in range(RPW // L):
    bb[pl.ds(v * L, L)] = startv[pl.ds(v * L, L)]

  def tbody(t, _):
    pltpu.sync_copy(bb, base.at[pl.ds(t * NPAD + c0, RPW)])
    for v in range(RPW // L):
      bb[pl.ds(v * L, L)] = (bb[pl.ds(v * L, L)] +
                             buf[pl.ds(t * RPW + v * L, L)])
    return 0

  lax.fori_loop(0, NW, tbody, 0)


@functools.partial(
    pl.kernel, mesh=_mesh, compiler_params=_CP,
    out_type=[
        jax.ShapeDtypeStruct((EP,), jnp.int32),
        jax.ShapeDtypeStruct((EP,), jnp.int32),
        jax.ShapeDtypeStruct((EP,), jnp.int32),
    ],
    scratch_types=[
        pltpu.VMEM((NPAD,), jnp.float32),
        pltpu.VMEM((EC,), jnp.int32),
        pltpu.VMEM((EC,), jnp.int32),
        pltpu.VMEM((2, ECH), jnp.int32),
        pltpu.VMEM((2, ECH), jnp.int32),
        pltpu.VMEM((2, ECH), jnp.int32),
        pltpu.VMEM((2, ECH), jnp.int32),
        pltpu.VMEM((L,), jnp.int32),
        pltpu.SemaphoreType.DMA,
    ])
def _place_k(srcr, dstr, base, s_src, s_dst, perm, bw, srca, dsta, posb,
             srb, drb, eib, zb, sem):
  w = _wid()
  ebase = w * EC
  pltpu.sync_copy(base.at[pl.ds(w * NPAD, NPAD)], bw)
  pltpu.sync_copy(srcr.at[pl.ds(ebase, EC)], srca)
  pltpu.sync_copy(dstr.at[pl.ds(ebase, EC)], dsta)
  it = _iota()
  ones = jnp.ones((L,), jnp.float32)

  def sc_copies(p):
    return (
        pltpu.make_async_copy(srb.at[p], s_src.at[posb.at[p]], sem),
        pltpu.make_async_copy(drb.at[p], s_dst.at[posb.at[p]], sem),
        pltpu.make_async_copy(eib.at[p], perm.at[posb.at[p]], sem),
    )

  def chunk(j, _):
    p = lax.rem(j, 2)

    @pl.when(j >= 2)
    def _():
      for c in sc_copies(p):
        c.wait()

    pv = jnp.full((L,), p, jnp.int32)
    for g in range(G):
      d16 = plsc.load_gather(dsta, [j * ECH + g * L + it])
      s16 = plsc.load_gather(srca, [j * ECH + g * L + it])
      kk, lane = plsc.sort_key_val(d16, it)
      vv = ones
      for sh in (1, 2, 4, 8):
        idx = jnp.maximum(it - sh, 0)
        kq = _gather16(kk, idx)
        vq = _gather16(vv, idx)
        eq = (it >= sh) & (kq == kk)
        vv = jnp.where(eq, vv + vq, vv)
      nxt = _gather16(kk, jnp.minimum(it + 1, L - 1))
      last = (it == L - 1) | (nxt != kk)
      cur = plsc.load_gather(bw, [kk])
      pos16 = (cur + vv - 1.0).astype(jnp.int32)
      plsc.store_scatter(bw, [kk], cur + vv, mask=last)
      src_s = _gather16(s16, lane)
      eid = ebase + j * ECH + g * L + lane
      cvec = g * L + it
      plsc.store_scatter(posb, [pv, cvec], pos16)
      plsc.store_scatter(srb, [pv, cvec], src_s)
      plsc.store_scatter(drb, [pv, cvec], kk)
      plsc.store_scatter(eib, [pv, cvec], eid)
    for c in sc_copies(p):
      c.start()
    return 0

  lax.fori_loop(0, NCHK, chunk, 0)
  for p in range(2):
    for c in sc_copies(p):
      c.wait()

  @pl.when(w == NW - 1)
  def _():
    zb[pl.ds(0, L)] = jnp.zeros((L,), jnp.int32)

    def padb(i, _):
      pltpu.sync_copy(zb, s_src.at[pl.ds(E + i * L, L)])
      pltpu.sync_copy(zb, s_dst.at[pl.ds(E + i * L, L)])
      pltpu.sync_copy(zb, perm.at[pl.ds(E + i * L, L)])
      return 0

    lax.fori_loop(0, (EP - E) // L, padb, 0)


@functools.partial(
    pl.kernel, mesh=_mesh, compiler_params=_CP,
    out_type=jax.ShapeDtypeStruct((NPAD, HID), jnp.float32),
    scratch_types=[
        pltpu.VMEM((RPW, HID), jnp.float32),
        pltpu.VMEM((2, ECH, HID), jnp.float32),
        pltpu.VMEM((2, ECH), jnp.int32),
        pltpu.VMEM((2, ECH), jnp.int32),
        pltpu.VMEM((2, ECH), jnp.int32),
        pltpu.VMEM((2, ECH), jnp.float32),
        pltpu.VMEM((RPW + 8,), jnp.float32),
        pltpu.SemaphoreType.DMA,
        pltpu.SemaphoreType.DMA,
    ])
def _aggcsr_k(alpha, perm, s_src, s_dst, start, xl, zeros, out, out_buf,
              rows, srcb, dstb, permb, alb, startv, srow, sidx):
  w = _wid()
  c0 = w * RPW
  pltpu.sync_copy(zeros.at[pl.ds(0, RPW)], out_buf)
  pltpu.sync_copy(start.at[pl.ds(c0, RPW + 8)], startv)
  it = _iota()
  lo = startv[pl.ds(0, L)][0].astype(jnp.int32)
  hi = startv[pl.ds(RPW - 8, L)][8].astype(jnp.int32)
  lo8 = pl.multiple_of(lo - lax.rem(lo, 8), 8)
  nch = (hi - lo8 + (ECH - 1)) // ECH

  def idx_copies(j, p):
    e0 = pl.multiple_of(lo8 + j * ECH, 8)
    return (
        pltpu.make_async_copy(s_src.at[pl.ds(e0, ECH)], srcb.at[p], sidx),
        pltpu.make_async_copy(s_dst.at[pl.ds(e0, ECH)], dstb.at[p], sidx),
        pltpu.make_async_copy(perm.at[pl.ds(e0, ECH)], permb.at[p], sidx),
    )

  def row_copies(j, p):
    return (
        pltpu.make_async_copy(xl.at[srcb.at[p]], rows.at[p], srow),
        pltpu.make_async_copy(alpha.at[permb.at[p]], alb.at[p], srow),
    )

  @pl.when(nch > 0)
  def _():
    for c in idx_copies(0, 0):
      c.start()
      c.wait()
    for c in row_copies(0, 0):
      c.start()

  def chunk(j, _):
    p = lax.rem(j, 2)

    @pl.when(j < nch - 1)
    def _():
      for c in idx_copies(j + 1, 1 - p):
        c.start()

    for c in row_copies(j, p):
      c.wait()

    pv = jnp.full((L,), p, jnp.int32)
    base_pos = lo8 + j * ECH
    for g in range(G):
      cvec = g * L + it
      p16 = base_pos + g * L + it
      valid = (p16 >= lo) & (p16 < hi)
      a16 = plsc.load_gather(alb, [pv, cvec])
      d16 = plsc.load_gather(dstb, [pv, cvec])
      a_eff = jnp.where(valid, a16, 0.0)
      dl_eff = jnp.where(valid, d16 - c0, 0)
      for lb in range(0, L, 4):
        xs = []
        for l in range(lb, lb + 4):
          rv = g * L + jnp.full((L,), l, jnp.int32)
          a_sc = a_eff[l]
          for cv in range(HID // L):
            cc = cv * L + it
            xs.append(plsc.load_gather(rows, [pv, rv, cc]) * a_sc)
        k = 0
        for l in range(lb, lb + 4):
          dlv = jnp.full((L,), 1, jnp.int32) * dl_eff[l]
          for cv in range(HID // L):
            cc = cv * L + it
            plsc.addupdate_scatter(out_buf, [dlv, cc], xs[k])
            k += 1

    @pl.when(j < nch - 1)
    def _():
      for c in idx_copies(j + 1, 1 - p):
        c.wait()
      for c in row_copies(j + 1, 1 - p):
        c.start()

    return 0

  lax.fori_loop(0, nch, chunk, 0)
  pltpu.sync_copy(out_buf, out.at[pl.ds(c0, RPW)])


_BR = 512


def _combine_body(g_ref, xp_ref, wa_ref, wb_ref, bp_ref, o_ref):
  acc = jnp.dot(g_ref[...], wa_ref[...], preferred_element_type=jnp.float32)
  acc += jnp.dot(xp_ref[...], wb_ref[...], preferred_element_type=jnp.float32)
  o_ref[...] = jnp.maximum(acc + bp_ref[...], 0.0)


def _tc_combine(g, xp, wa, wb, bp):
  return pl.pallas_call(
      _combine_body,
      grid=(NPAD // _BR,),
      in_specs=[
          pl.BlockSpec((_BR, HID), lambda i: (i, 0)),
          pl.BlockSpec((_BR, FEAT), lambda i: (i, 0)),
          pl.BlockSpec((HID, HID), lambda i: (0, 0)),
          pl.BlockSpec((FEAT, HID), lambda i: (0, 0)),
          pl.BlockSpec((1, HID), lambda i: (0, 0)),
      ],
      out_specs=pl.BlockSpec((_BR, HID), lambda i: (i, 0)),
      out_shape=jax.ShapeDtypeStruct((NPAD, HID), jnp.float32),
  )(g, xp, wa, wb, bp)


def _proj_body(x_ref, w_ref, o_ref):
  o_ref[0] = jnp.dot(x_ref[...], w_ref[...],
                     preferred_element_type=jnp.float32)


def _tc_proj_heads(x, w, heads):
  return pl.pallas_call(
      _proj_body,
      grid=(heads, NPAD // _BR),
      in_specs=[
          pl.BlockSpec((_BR, HID), lambda h, i: (i, 0)),
          pl.BlockSpec((HID, HID), lambda h, i: (0, h)),
      ],
      out_specs=pl.BlockSpec((1, _BR, HID), lambda h, i: (h, i, 0)),
      out_shape=jax.ShapeDtypeStruct((heads, NPAD, HID), jnp.float32),
  )(x, w)


def _layer2in_body(p_ref, b1_ref, wl_ref, wr_ref, ol_ref, or_ref):
  accl = jnp.zeros((_BR, HID), jnp.float32)
  accr = jnp.zeros((_BR, HID), jnp.float32)
  for h in range(4):
    xh = jnp.maximum(p_ref[h] + b1_ref[0, pl.ds(h * HID, HID)], 0.0)
    accl += jnp.dot(xh, wl_ref[pl.ds(h * HID, HID), :],
                    preferred_element_type=jnp.float32)
    accr += jnp.dot(xh, wr_ref[pl.ds(h * HID, HID), :],
                    preferred_element_type=jnp.float32)
  ol_ref[...] = accl
  or_ref[...] = accr


def _tc_layer2in(p, b1, wl, wr):
  return pl.pallas_call(
      _layer2in_body,
      grid=(NPAD // _BR,),
      in_specs=[
          pl.BlockSpec((4, _BR, HID), lambda i: (0, i, 0)),
          pl.BlockSpec((1, 4 * HID), lambda i: (0, 0)),
          pl.BlockSpec((4 * HID, HID), lambda i: (0, 0)),
          pl.BlockSpec((4 * HID, HID), lambda i: (0, 0)),
      ],
      out_specs=[
          pl.BlockSpec((_BR, HID), lambda i: (i, 0)),
          pl.BlockSpec((_BR, HID), lambda i: (i, 0)),
      ],
      out_shape=[
          jax.ShapeDtypeStruct((NPAD, HID), jnp.float32),
          jax.ShapeDtypeStruct((NPAD, HID), jnp.float32),
      ],
  )(p, b1, wl, wr)


def _final_body(p_ref, b2_ref, wo_ref, bo_ref, o_ref):
  xo = p_ref[...] + b2_ref[...]
  o_ref[...] = jnp.dot(xo, wo_ref[...],
                       preferred_element_type=jnp.float32) + bo_ref[...]


def _tc_final(p, b2, wo, bo):
  return pl.pallas_call(
      _final_body,
      grid=(NPAD // _BR,),
      in_specs=[
          pl.BlockSpec((_BR, HID), lambda i: (i, 0)),
          pl.BlockSpec((1, HID), lambda i: (0, 0)),
          pl.BlockSpec((HID, HID), lambda i: (0, 0)),
          pl.BlockSpec((1, HID), lambda i: (0, 0)),
      ],
      out_specs=pl.BlockSpec((_BR, HID), lambda i: (i, 0)),
      out_shape=jax.ShapeDtypeStruct((NPAD, HID), jnp.float32),
  )(p, b2, wo, bo)


def _gat_head(xl_h, xr_h, att_h, src, dst, zeros, csr):
  s_src, s_dst, perm, start = csr
  score = _score_k(xl_h, xr_h, att_h, src, dst)
  m_priv = _segmax_k(score, dst)
  m_glob = _combine_max(m_priv)
  ex, s_priv = _expsum_k(score, dst, m_glob)
  s_glob = _combine_add(s_priv)
  alpha = _alpha_k(ex, dst, s_glob)
  return _aggcsr_k(alpha, perm, s_src, s_dst, start, xl_h, zeros)


def kernel(x_player, edge_index, emb, Wp, bp, Wl1, Wr1, att1, b1, Wl2, Wr2,
           att2, b2, Wo, bo):
  src = edge_index[0].astype(jnp.int32)
  dst = edge_index[1].astype(jnp.int32)

  ids = x_player[:, 0].astype(jnp.int32)
  ids_pad = jnp.pad(ids, (0, NPAD - N))
  xp = jnp.pad(x_player[:, 1:], ((0, NPAD - N), (0, 0)))

  g = _emb_gather(emb, ids_pad)
  x0 = _tc_combine(g, xp, Wp[:HID], Wp[HID:], bp.reshape(1, HID))

  xl1 = _tc_proj_heads(x0, Wl1, 4)
  xr1 = _tc_proj_heads(x0, Wr1, 4)
  zeros = jnp.zeros((NPAD, HID), jnp.float32)
  hist = _hist_k(dst)
  ssum = _slicesum_k(hist)
  base, start = _base_k(hist, ssum)
  s_src, s_dst, perm = _place_k(src, dst, base)
  csr = (s_src, s_dst, perm, start)
  parts1 = [_gat_head(xl1[h], xr1[h], att1[h], src, dst, zeros, csr)
            for h in range(4)]
  p1 = jnp.stack(parts1, axis=0)

  xl2, xr2 = _tc_layer2in(p1, b1.reshape(1, 4 * HID), Wl2, Wr2)
  p2 = _gat_head(xl2, xr2, att2[0], src, dst, zeros, csr)

  wo_pad = jnp.pad(Wo, ((0, 0), (0, HID - OUTD)))
  bo_pad = jnp.pad(bo, (0, HID - OUTD)).reshape(1, HID)
  y = _tc_final(p2, b2.reshape(1, HID), wo_pad, bo_pad)
  return y[:N, :OUTD]
